# Initial kernel scaffold; baseline (speedup 1.0000x reference)
import jax
import jax.numpy as jnp
from jax.experimental import pallas as pl


def kernel(x, edge_index, batch, W0, b0, W1, b1, gamma, beta, lin2_W, lin2_b):
    raise NotImplementedError("write your pallas kernel here")


# speedup vs baseline: 6.1706x; 6.1706x over previous
import functools

import jax
import jax.numpy as jnp
from jax import lax
from jax.experimental import pallas as pl
from jax.experimental.pallas import tpu as pltpu
from jax.experimental.pallas import tpu_sc as plsc

N = 10000
E = 320000
NUM_PROP = 6
N_GRAPHS = 64

NC = 2
NS = 16
NW = NC * NS
CHUNK = 128
N_PAD = 10112
EPW = 10112
E_PAD = NW * EPW
N_CHUNKS = EPW // CHUNK
DEG_D = 16

_MESH = plsc.VectorSubcoreMesh(core_axis_name="c", subcore_axis_name="s")


def _make_spmv(d):
    rows_per_sub = N_PAD // NS

    @functools.partial(
        pl.kernel, mesh=_MESH,
        compiler_params=pltpu.CompilerParams(use_tc_tiling_on_sc=False),
        out_type=jax.ShapeDtypeStruct((NC, N_PAD, d), jnp.float32),
        scratch_types=[
            pltpu.VMEM((N_CHUNKS, CHUNK), jnp.int32),
            pltpu.VMEM((N_CHUNKS, CHUNK), jnp.int32),
            pltpu.VMEM((CHUNK, d), jnp.float32),
            pltpu.VMEM_SHARED((N_PAD, d), jnp.float32),
            pltpu.SemaphoreType.DMA,
        ],
    )
    def spmv(table, srcs, dsts, zeros, out, src_v, dst_v, rows_v, acc, sem):
        c = lax.axis_index("c")
        s = lax.axis_index("s")
        w = c * NS + s
        r0 = s * rows_per_sub
        pltpu.sync_copy(zeros.at[pl.ds(r0, rows_per_sub)],
                        acc.at[pl.ds(r0, rows_per_sub)])
        pltpu.sync_copy(srcs.at[w], src_v)
        pltpu.sync_copy(dsts.at[w], dst_v)
        plsc.subcore_barrier()

        def body(k, carry):
            pltpu.async_copy(table.at[src_v.at[k]], rows_v, sem).wait()
            pltpu.sync_copy(rows_v, acc.at[dst_v.at[k]], add=True)
            return carry

        lax.fori_loop(0, N_CHUNKS, body, 0)
        plsc.subcore_barrier()
        pltpu.sync_copy(acc.at[pl.ds(r0, rows_per_sub)],
                        out.at[c, pl.ds(r0, rows_per_sub)])

    return spmv


def _make_degree():
    rows_per_sub = N_PAD // NS

    @functools.partial(
        pl.kernel, mesh=_MESH,
        compiler_params=pltpu.CompilerParams(use_tc_tiling_on_sc=False),
        out_type=jax.ShapeDtypeStruct((NC, N_PAD, DEG_D), jnp.float32),
        scratch_types=[
            pltpu.VMEM((N_CHUNKS, CHUNK), jnp.int32),
            pltpu.VMEM((CHUNK, DEG_D), jnp.float32),
            pltpu.VMEM_SHARED((N_PAD, DEG_D), jnp.float32),
        ],
    )
    def degk(ones, idxs, zeros, out, idx_v, ones_v, acc):
        c = lax.axis_index("c")
        s = lax.axis_index("s")
        w = c * NS + s
        r0 = s * rows_per_sub
        pltpu.sync_copy(zeros.at[pl.ds(r0, rows_per_sub)],
                        acc.at[pl.ds(r0, rows_per_sub)])
        pltpu.sync_copy(idxs.at[w], idx_v)
        pltpu.sync_copy(ones, ones_v)
        plsc.subcore_barrier()

        def body(k, carry):
            pltpu.sync_copy(ones_v, acc.at[idx_v.at[k]], add=True)
            return carry

        lax.fori_loop(0, N_CHUNKS, body, 0)
        plsc.subcore_barrier()
        pltpu.sync_copy(acc.at[pl.ds(r0, rows_per_sub)],
                        out.at[c, pl.ds(r0, rows_per_sub)])

    return degk


def _prep_body(degp_ref, x_ref, dis_ref, s0_ref):
    d = degp_ref[0, :, 0:1] + degp_ref[1, :, 0:1]
    dis = jnp.where(d > 0.0, lax.rsqrt(jnp.maximum(d, 1.0)), 0.0)
    dis_ref[...] = dis
    s0_ref[...] = x_ref[...] * dis[:N]


def _step1_body(accp_ref, dis_ref, tx_ref, s_ref):
    dis = dis_ref[...][:N]
    a = accp_ref[0, :N] + accp_ref[1, :N]
    t = -dis * a
    tx_ref[...] = t
    s_ref[...] = dis * t


def _stepk_body(accp_ref, dis_ref, txprev_ref, tx_ref, s_ref):
    dis = dis_ref[...][:N]
    a = accp_ref[0, :N] + accp_ref[1, :N]
    t = -2.0 * dis * a - txprev_ref[...]
    tx_ref[...] = t
    s_ref[...] = dis * t


def _post_body(accp_ref, dis_ref, tx0_ref, tx1_ref, tx2_ref, tx3_ref,
               w_ref, b_ref, gamma_ref, beta_ref, h_ref, s_ref):
    dis = dis_ref[...][:N]
    tx4 = -2.0 * dis * (accp_ref[0, :N] + accp_ref[1, :N]) - tx2_ref[...]
    out = jnp.dot(tx0_ref[...], w_ref[0], preferred_element_type=jnp.float32)
    out += jnp.dot(tx1_ref[...], w_ref[1], preferred_element_type=jnp.float32)
    out += jnp.dot(tx2_ref[...], w_ref[2], preferred_element_type=jnp.float32)
    out += jnp.dot(tx3_ref[...], w_ref[3], preferred_element_type=jnp.float32)
    out += jnp.dot(tx4, w_ref[4], preferred_element_type=jnp.float32)
    h0 = jnp.maximum(out + b_ref[...], 0.0)
    mu = jnp.mean(h0, axis=0, keepdims=True)
    var = jnp.mean(h0 * h0, axis=0, keepdims=True) - mu * mu
    h = (h0 - mu) * lax.rsqrt(var + 1e-5) * gamma_ref[...] + beta_ref[...]
    h_ref[...] = h
    s_ref[...] = dis * h


def _pool_body(h_ref, batch_ref, w_ref, b_ref, out_ref):
    gids = lax.broadcasted_iota(jnp.int32, (N_GRAPHS, N), 0)
    m = (gids == batch_ref[...]).astype(jnp.float32)
    sums = jnp.dot(m, h_ref[...], preferred_element_type=jnp.float32)
    counts = jnp.sum(m, axis=1, keepdims=True)
    pooled = sums / jnp.clip(counts, 1.0, None)
    out_ref[...] = (jnp.dot(pooled, w_ref[...],
                            preferred_element_type=jnp.float32) + b_ref[...])


def _tc(body, out_shapes, *args):
    return pl.pallas_call(
        body, out_shape=out_shapes,
        compiler_params=pltpu.CompilerParams(
            vmem_limit_bytes=100 * 1024 * 1024),
    )(*args)


def kernel(x, edge_index, batch, W0, b0, W1, b1, gamma, beta, lin2_W, lin2_b):
    f32 = jnp.float32
    src = edge_index[0]
    dst = edge_index[1]
    pad = E_PAD - E
    src_g = jnp.concatenate([src, jnp.zeros((pad,), jnp.int32)]).reshape(
        NW, N_CHUNKS, CHUNK)
    src_s = jnp.concatenate([src, jnp.full((pad,), N, jnp.int32)]).reshape(
        NW, N_CHUNKS, CHUNK)
    dst_s = jnp.concatenate([dst, jnp.full((pad,), N, jnp.int32)]).reshape(
        NW, N_CHUNKS, CHUNK)

    zeros_deg = jnp.zeros((N_PAD, DEG_D), f32)
    zeros64 = jnp.zeros((N_PAD, 64), f32)
    zeros128 = jnp.zeros((N_PAD, 128), f32)
    ones_deg = jnp.ones((CHUNK, DEG_D), f32)

    degree = _make_degree()
    spmv128 = _make_spmv(128)
    spmv64 = _make_spmv(64)

    degp = degree(ones_deg, src_s, zeros_deg)
    dis, s = _tc(_prep_body,
                 (jax.ShapeDtypeStruct((N_PAD, 1), f32),
                  jax.ShapeDtypeStruct((N, 128), f32)),
                 degp, x)

    b0_ = b0.reshape(1, -1)
    b1_ = b1.reshape(1, -1)
    gamma_ = gamma.reshape(1, -1)
    beta_ = beta.reshape(1, -1)

    h = x
    for layer in range(1 + NUM_PROP):
        d_in = 128 if layer == 0 else 64
        spmv = spmv128 if layer == 0 else spmv64
        zeros = zeros128 if layer == 0 else zeros64
        W = W0 if layer == 0 else W1
        b_ = b0_ if layer == 0 else b1_
        sds = jax.ShapeDtypeStruct((N, d_in), f32)

        accp = spmv(s, src_g, dst_s, zeros)
        tx1, s = _tc(_step1_body, (sds, sds), accp, dis)
        accp = spmv(s, src_g, dst_s, zeros)
        tx2, s = _tc(_stepk_body, (sds, sds), accp, dis, h)
        accp = spmv(s, src_g, dst_s, zeros)
        tx3, s = _tc(_stepk_body, (sds, sds), accp, dis, tx1)
        accp = spmv(s, src_g, dst_s, zeros)
        h, s = _tc(_post_body,
                   (jax.ShapeDtypeStruct((N, 64), f32),
                    jax.ShapeDtypeStruct((N, 64), f32)),
                   accp, dis, h, tx1, tx2, tx3, W, b_, gamma_, beta_)

    return _tc(_pool_body, jax.ShapeDtypeStruct((N_GRAPHS, 10), f32),
               h, batch.reshape(1, N), lin2_W, lin2_b.reshape(1, -1))


# speedup vs baseline: 7.6416x; 1.2384x over previous
import functools

import jax
import jax.numpy as jnp
from jax import lax
from jax.experimental import pallas as pl
from jax.experimental.pallas import tpu as pltpu
from jax.experimental.pallas import tpu_sc as plsc

N = 10000
E = 320000
NUM_PROP = 6
N_GRAPHS = 64

NC = 2
NS = 16
NW = NC * NS
CHUNK = 128
N_PAD = 10112
EPW = 10112
E_PAD = NW * EPW
N_CHUNKS = EPW // CHUNK
DEG_D = 16

_MESH = plsc.VectorSubcoreMesh(core_axis_name="c", subcore_axis_name="s")


def _make_spmv(d, chunk):
    rows_per_sub = N_PAD // NS
    n_chunks = EPW // chunk

    nfull = n_chunks // 4
    assert nfull >= 2

    @functools.partial(
        pl.kernel, mesh=_MESH,
        compiler_params=pltpu.CompilerParams(use_tc_tiling_on_sc=False),
        out_type=jax.ShapeDtypeStruct((NC, N_PAD, d), jnp.float32),
        scratch_types=[
            pltpu.VMEM((n_chunks, chunk), jnp.int32),
            pltpu.VMEM((n_chunks, chunk), jnp.int32),
            pltpu.VMEM((chunk, d), jnp.float32),
            pltpu.VMEM((chunk, d), jnp.float32),
            pltpu.VMEM((chunk, d), jnp.float32),
            pltpu.VMEM((chunk, d), jnp.float32),
            pltpu.VMEM_SHARED((N_PAD, d), jnp.float32),
        ] + [pltpu.SemaphoreType.DMA] * 8,
    )
    def spmv(table, srcs, dsts, zeros, out,
             src_v, dst_v, b0, b1, b2, b3, acc,
             sg0, sg1, sg2, sg3, ss0, ss1, ss2, ss3):
        bufs = [b0, b1, b2, b3]
        sgs = [sg0, sg1, sg2, sg3]
        sss = [ss0, ss1, ss2, ss3]
        c = lax.axis_index("c")
        s = lax.axis_index("s")
        w = c * NS + s
        r0 = s * rows_per_sub
        pltpu.sync_copy(zeros.at[pl.ds(r0, rows_per_sub)],
                        acc.at[pl.ds(r0, rows_per_sub)])
        pltpu.sync_copy(srcs.at[w], src_v)
        pltpu.sync_copy(dsts.at[w], dst_v)
        plsc.subcore_barrier()

        def gstart(k, i):
            pltpu.async_copy(table.at[src_v.at[k]], bufs[i], sgs[i])

        def gwait(k, i):
            pltpu.make_async_copy(table.at[src_v.at[k]], bufs[i],
                                  sgs[i]).wait()

        def sstart(k, i):
            pltpu.async_copy(bufs[i], acc.at[dst_v.at[k]], sss[i], add=True)

        def swait(k, i):
            pltpu.make_async_copy(bufs[i], acc.at[dst_v.at[k]],
                                  sss[i]).wait()

        def static_step(k):
            i = k % 4
            gwait(k, i)
            sstart(k, i)
            k2 = k + 2
            if k2 < n_chunks:
                if k2 - 4 >= 0:
                    swait(k2 - 4, k2 % 4)
                gstart(k2, k2 % 4)

        gstart(0, 0)
        gstart(1, 1)
        for k in range(4):
            static_step(k)

        def group(j, carry):
            k0 = 4 * j
            for i in range(4):
                k = k0 + i
                gwait(k, i)
                sstart(k, i)
                swait(k - 2, (i + 2) % 4)
                gstart(k + 2, (i + 2) % 4)
            return carry

        lax.fori_loop(1, nfull - 1, group, 0)
        for k in range(4 * (nfull - 1), n_chunks):
            static_step(k)
        for k in range(max(0, n_chunks - 4), n_chunks):
            swait(k, k % 4)
        plsc.subcore_barrier()
        pltpu.sync_copy(acc.at[pl.ds(r0, rows_per_sub)],
                        out.at[c, pl.ds(r0, rows_per_sub)])

    return spmv


def _make_degree():
    rows_per_sub = N_PAD // NS

    @functools.partial(
        pl.kernel, mesh=_MESH,
        compiler_params=pltpu.CompilerParams(use_tc_tiling_on_sc=False),
        out_type=jax.ShapeDtypeStruct((NC, N_PAD, DEG_D), jnp.float32),
        scratch_types=[
            pltpu.VMEM((N_CHUNKS, CHUNK), jnp.int32),
            pltpu.VMEM((CHUNK, DEG_D), jnp.float32),
            pltpu.VMEM_SHARED((N_PAD, DEG_D), jnp.float32),
            pltpu.SemaphoreType.DMA,
        ],
    )
    def degk(ones, idxs, zeros, out, idx_v, ones_v, acc, ssem):
        c = lax.axis_index("c")
        s = lax.axis_index("s")
        w = c * NS + s
        r0 = s * rows_per_sub
        pltpu.sync_copy(zeros.at[pl.ds(r0, rows_per_sub)],
                        acc.at[pl.ds(r0, rows_per_sub)])
        pltpu.sync_copy(idxs.at[w], idx_v)
        pltpu.sync_copy(ones, ones_v)
        plsc.subcore_barrier()

        def body(k, carry):
            pltpu.async_copy(ones_v, acc.at[idx_v.at[k]], ssem, add=True)
            return carry

        lax.fori_loop(0, N_CHUNKS, body, 0)

        def drain(k, carry):
            pltpu.make_async_copy(ones_v, acc.at[idx_v.at[k]], ssem).wait()
            return carry

        lax.fori_loop(0, N_CHUNKS, drain, 0)
        plsc.subcore_barrier()
        pltpu.sync_copy(acc.at[pl.ds(r0, rows_per_sub)],
                        out.at[c, pl.ds(r0, rows_per_sub)])

    return degk


def _prep_body(degp_ref, x_ref, dis_ref, s0_ref):
    d = degp_ref[0, :, 0:1] + degp_ref[1, :, 0:1]
    dis = jnp.where(d > 0.0, lax.rsqrt(jnp.maximum(d, 1.0)), 0.0)
    dis_ref[...] = dis
    s0_ref[...] = x_ref[...] * dis[:N]


def _step1_body(accp_ref, dis_ref, tx_ref, s_ref):
    dis = dis_ref[...][:N]
    a = accp_ref[0, :N] + accp_ref[1, :N]
    t = -dis * a
    tx_ref[...] = t
    s_ref[...] = dis * t


def _stepk_body(accp_ref, dis_ref, txprev_ref, tx_ref, s_ref):
    dis = dis_ref[...][:N]
    a = accp_ref[0, :N] + accp_ref[1, :N]
    t = -2.0 * dis * a - txprev_ref[...]
    tx_ref[...] = t
    s_ref[...] = dis * t


def _post_body(accp_ref, dis_ref, tx0_ref, tx1_ref, tx2_ref, tx3_ref,
               w_ref, b_ref, gamma_ref, beta_ref, h_ref, s_ref):
    dis = dis_ref[...][:N]
    tx4 = -2.0 * dis * (accp_ref[0, :N] + accp_ref[1, :N]) - tx2_ref[...]
    out = jnp.dot(tx0_ref[...], w_ref[0], preferred_element_type=jnp.float32)
    out += jnp.dot(tx1_ref[...], w_ref[1], preferred_element_type=jnp.float32)
    out += jnp.dot(tx2_ref[...], w_ref[2], preferred_element_type=jnp.float32)
    out += jnp.dot(tx3_ref[...], w_ref[3], preferred_element_type=jnp.float32)
    out += jnp.dot(tx4, w_ref[4], preferred_element_type=jnp.float32)
    h0 = jnp.maximum(out + b_ref[...], 0.0)
    mu = jnp.mean(h0, axis=0, keepdims=True)
    hc = h0 - mu
    var = jnp.mean(hc * hc, axis=0, keepdims=True)
    h = (h0 - mu) * lax.rsqrt(var + 1e-5) * gamma_ref[...] + beta_ref[...]
    h_ref[...] = h
    s_ref[...] = dis * h


def _pool_body(h_ref, batch_ref, w_ref, b_ref, out_ref):
    gids = lax.broadcasted_iota(jnp.int32, (N_GRAPHS, N), 0)
    m = (gids == batch_ref[...]).astype(jnp.float32)
    sums = jnp.dot(m, h_ref[...], preferred_element_type=jnp.float32)
    counts = jnp.sum(m, axis=1, keepdims=True)
    pooled = sums / jnp.clip(counts, 1.0, None)
    out_ref[...] = (jnp.dot(pooled, w_ref[...],
                            preferred_element_type=jnp.float32) + b_ref[...])


def _tc(body, out_shapes, *args):
    return pl.pallas_call(
        body, out_shape=out_shapes,
        compiler_params=pltpu.CompilerParams(
            vmem_limit_bytes=100 * 1024 * 1024),
    )(*args)


def kernel(x, edge_index, batch, W0, b0, W1, b1, gamma, beta, lin2_W, lin2_b):
    f32 = jnp.float32
    src = edge_index[0]
    dst = edge_index[1]
    pad = E_PAD - E
    src_g = jnp.concatenate([src, jnp.zeros((pad,), jnp.int32)])
    src_s = jnp.concatenate([src, jnp.full((pad,), N, jnp.int32)]).reshape(
        NW, N_CHUNKS, CHUNK)
    dst_s = jnp.concatenate([dst, jnp.full((pad,), N, jnp.int32)])
    src_g128, dst_s128 = (a.reshape(NW, EPW // 32, 32)
                          for a in (src_g, dst_s))
    src_g64, dst_s64 = (a.reshape(NW, N_CHUNKS, CHUNK)
                        for a in (src_g, dst_s))

    zeros_deg = jnp.zeros((N_PAD, DEG_D), f32)
    zeros64 = jnp.zeros((N_PAD, 64), f32)
    zeros128 = jnp.zeros((N_PAD, 128), f32)
    ones_deg = jnp.ones((CHUNK, DEG_D), f32)

    degree = _make_degree()
    spmv128 = _make_spmv(128, 32)
    spmv64 = _make_spmv(64, 128)

    degp = degree(ones_deg, src_s, zeros_deg)
    dis, s = _tc(_prep_body,
                 (jax.ShapeDtypeStruct((N_PAD, 1), f32),
                  jax.ShapeDtypeStruct((N, 128), f32)),
                 degp, x)

    b0_ = b0.reshape(1, -1)
    b1_ = b1.reshape(1, -1)
    gamma_ = gamma.reshape(1, -1)
    beta_ = beta.reshape(1, -1)

    h = x
    for layer in range(1 + NUM_PROP):
        d_in = 128 if layer == 0 else 64
        spmv = spmv128 if layer == 0 else spmv64
        src_g_r = src_g128 if layer == 0 else src_g64
        dst_s_r = dst_s128 if layer == 0 else dst_s64
        zeros = zeros128 if layer == 0 else zeros64
        W = W0 if layer == 0 else W1
        b_ = b0_ if layer == 0 else b1_
        sds = jax.ShapeDtypeStruct((N, d_in), f32)

        accp = spmv(s, src_g_r, dst_s_r, zeros)
        tx1, s = _tc(_step1_body, (sds, sds), accp, dis)
        accp = spmv(s, src_g_r, dst_s_r, zeros)
        tx2, s = _tc(_stepk_body, (sds, sds), accp, dis, h)
        accp = spmv(s, src_g_r, dst_s_r, zeros)
        tx3, s = _tc(_stepk_body, (sds, sds), accp, dis, tx1)
        accp = spmv(s, src_g_r, dst_s_r, zeros)
        h, s = _tc(_post_body,
                   (jax.ShapeDtypeStruct((N, 64), f32),
                    jax.ShapeDtypeStruct((N, 64), f32)),
                   accp, dis, h, tx1, tx2, tx3, W, b_, gamma_, beta_)

    return _tc(_pool_body, jax.ShapeDtypeStruct((N_GRAPHS, 10), f32),
               h, batch.reshape(1, N), lin2_W, lin2_b.reshape(1, -1))


# speedup vs baseline: 7.7679x; 1.0165x over previous
import functools

import jax
import jax.numpy as jnp
from jax import lax
from jax.experimental import pallas as pl
from jax.experimental.pallas import tpu as pltpu
from jax.experimental.pallas import tpu_sc as plsc

N = 10000
E = 320000
NUM_PROP = 6
N_GRAPHS = 64

NC = 2
NS = 16
NW = NC * NS
CHUNK = 128
N_PAD = 10112
EPW = 10112
E_PAD = NW * EPW
N_CHUNKS = EPW // CHUNK
DEG_D = 16

_MESH = plsc.VectorSubcoreMesh(core_axis_name="c", subcore_axis_name="s")


def _make_spmv(d, chunk, RING=8, PREF=4):
    rows_per_sub = N_PAD // NS
    n_chunks = EPW // chunk

    nfull = n_chunks // RING
    assert nfull >= 2

    @functools.partial(
        pl.kernel, mesh=_MESH,
        compiler_params=pltpu.CompilerParams(use_tc_tiling_on_sc=False),
        out_type=jax.ShapeDtypeStruct((NC, N_PAD, d), jnp.float32),
        scratch_types=[
            pltpu.VMEM((n_chunks, chunk), jnp.int32),
            pltpu.VMEM((n_chunks, chunk), jnp.int32),
        ] + [pltpu.VMEM((chunk, d), jnp.float32)] * RING + [
            pltpu.VMEM_SHARED((N_PAD, d), jnp.float32),
        ] + [pltpu.SemaphoreType.DMA] * (2 * RING),
    )
    def spmv(table, srcs, dsts, zeros, out, src_v, dst_v, *rest):
        bufs = rest[:RING]
        acc = rest[RING]
        sgs = rest[RING + 1:RING + 1 + RING]
        sss = rest[RING + 1 + RING:]
        c = lax.axis_index("c")
        s = lax.axis_index("s")
        w = c * NS + s
        r0 = s * rows_per_sub
        pltpu.sync_copy(zeros.at[pl.ds(r0, rows_per_sub)],
                        acc.at[pl.ds(r0, rows_per_sub)])
        pltpu.sync_copy(srcs.at[w], src_v)
        pltpu.sync_copy(dsts.at[w], dst_v)
        plsc.subcore_barrier()

        def gstart(k, i):
            pltpu.async_copy(table.at[src_v.at[k]], bufs[i], sgs[i])

        def gwait(k, i):
            pltpu.make_async_copy(table.at[src_v.at[k]], bufs[i],
                                  sgs[i]).wait()

        def sstart(k, i):
            pltpu.async_copy(bufs[i], acc.at[dst_v.at[k]], sss[i], add=True)

        def swait(k, i):
            pltpu.make_async_copy(bufs[i], acc.at[dst_v.at[k]],
                                  sss[i]).wait()

        def static_step(k):
            i = k % RING
            gwait(k, i)
            sstart(k, i)
            k2 = k + PREF
            if k2 < n_chunks:
                if k2 - RING >= 0:
                    swait(k2 - RING, k2 % RING)
                gstart(k2, k2 % RING)

        for k in range(PREF):
            gstart(k, k)
        for k in range(RING):
            static_step(k)

        def group(j, carry):
            k0 = RING * j
            for i in range(RING):
                k = k0 + i
                gwait(k, i)
                sstart(k, i)
                swait(k - (RING - PREF), (i + PREF) % RING)
                gstart(k + PREF, (i + PREF) % RING)
            return carry

        lax.fori_loop(1, nfull - 1, group, 0)
        for k in range(RING * (nfull - 1), n_chunks):
            static_step(k)
        for k in range(max(0, n_chunks - RING), n_chunks):
            swait(k, k % RING)
        plsc.subcore_barrier()
        pltpu.sync_copy(acc.at[pl.ds(r0, rows_per_sub)],
                        out.at[c, pl.ds(r0, rows_per_sub)])

    return spmv


def _make_degree():
    rows_per_sub = N_PAD // NS

    @functools.partial(
        pl.kernel, mesh=_MESH,
        compiler_params=pltpu.CompilerParams(use_tc_tiling_on_sc=False),
        out_type=jax.ShapeDtypeStruct((NC, N_PAD, DEG_D), jnp.float32),
        scratch_types=[
            pltpu.VMEM((N_CHUNKS, CHUNK), jnp.int32),
            pltpu.VMEM((CHUNK, DEG_D), jnp.float32),
            pltpu.VMEM_SHARED((N_PAD, DEG_D), jnp.float32),
            pltpu.SemaphoreType.DMA,
        ],
    )
    def degk(ones, idxs, zeros, out, idx_v, ones_v, acc, ssem):
        c = lax.axis_index("c")
        s = lax.axis_index("s")
        w = c * NS + s
        r0 = s * rows_per_sub
        pltpu.sync_copy(zeros.at[pl.ds(r0, rows_per_sub)],
                        acc.at[pl.ds(r0, rows_per_sub)])
        pltpu.sync_copy(idxs.at[w], idx_v)
        pltpu.sync_copy(ones, ones_v)
        plsc.subcore_barrier()

        def body(k, carry):
            pltpu.async_copy(ones_v, acc.at[idx_v.at[k]], ssem, add=True)
            return carry

        lax.fori_loop(0, N_CHUNKS, body, 0)

        def drain(k, carry):
            pltpu.make_async_copy(ones_v, acc.at[idx_v.at[k]], ssem).wait()
            return carry

        lax.fori_loop(0, N_CHUNKS, drain, 0)
        plsc.subcore_barrier()
        pltpu.sync_copy(acc.at[pl.ds(r0, rows_per_sub)],
                        out.at[c, pl.ds(r0, rows_per_sub)])

    return degk


def _prep_body(degp_ref, x_ref, dis_ref, s0_ref):
    d = degp_ref[0, :, 0:1] + degp_ref[1, :, 0:1]
    dis = jnp.where(d > 0.0, lax.rsqrt(jnp.maximum(d, 1.0)), 0.0)
    dis_ref[...] = dis
    s0_ref[...] = x_ref[...] * dis[:N]


def _step1_body(accp_ref, dis_ref, tx_ref, s_ref):
    dis = dis_ref[...][:N]
    a = accp_ref[0, :N] + accp_ref[1, :N]
    t = -dis * a
    tx_ref[...] = t
    s_ref[...] = dis * t


def _stepk_body(accp_ref, dis_ref, txprev_ref, tx_ref, s_ref):
    dis = dis_ref[...][:N]
    a = accp_ref[0, :N] + accp_ref[1, :N]
    t = -2.0 * dis * a - txprev_ref[...]
    tx_ref[...] = t
    s_ref[...] = dis * t


def _post_body(accp_ref, dis_ref, tx0_ref, tx1_ref, tx2_ref, tx3_ref,
               w_ref, b_ref, gamma_ref, beta_ref, h_ref, s_ref):
    dis = dis_ref[...][:N]
    tx4 = -2.0 * dis * (accp_ref[0, :N] + accp_ref[1, :N]) - tx2_ref[...]
    out = jnp.dot(tx0_ref[...], w_ref[0], preferred_element_type=jnp.float32)
    out += jnp.dot(tx1_ref[...], w_ref[1], preferred_element_type=jnp.float32)
    out += jnp.dot(tx2_ref[...], w_ref[2], preferred_element_type=jnp.float32)
    out += jnp.dot(tx3_ref[...], w_ref[3], preferred_element_type=jnp.float32)
    out += jnp.dot(tx4, w_ref[4], preferred_element_type=jnp.float32)
    h0 = jnp.maximum(out + b_ref[...], 0.0)
    mu = jnp.mean(h0, axis=0, keepdims=True)
    hc = h0 - mu
    var = jnp.mean(hc * hc, axis=0, keepdims=True)
    h = (h0 - mu) * lax.rsqrt(var + 1e-5) * gamma_ref[...] + beta_ref[...]
    h_ref[...] = h
    s_ref[...] = dis * h


def _pool_body(h_ref, batch_ref, w_ref, b_ref, out_ref):
    gids = lax.broadcasted_iota(jnp.int32, (N_GRAPHS, N), 0)
    m = (gids == batch_ref[...]).astype(jnp.float32)
    sums = jnp.dot(m, h_ref[...], preferred_element_type=jnp.float32)
    counts = jnp.sum(m, axis=1, keepdims=True)
    pooled = sums / jnp.clip(counts, 1.0, None)
    out_ref[...] = (jnp.dot(pooled, w_ref[...],
                            preferred_element_type=jnp.float32) + b_ref[...])


def _tc(body, out_shapes, *args):
    return pl.pallas_call(
        body, out_shape=out_shapes,
        compiler_params=pltpu.CompilerParams(
            vmem_limit_bytes=100 * 1024 * 1024),
    )(*args)


def kernel(x, edge_index, batch, W0, b0, W1, b1, gamma, beta, lin2_W, lin2_b):
    f32 = jnp.float32
    src = edge_index[0]
    dst = edge_index[1]
    pad = E_PAD - E
    src_g = jnp.concatenate([src, jnp.zeros((pad,), jnp.int32)])
    src_s = jnp.concatenate([src, jnp.full((pad,), N, jnp.int32)]).reshape(
        NW, N_CHUNKS, CHUNK)
    dst_s = jnp.concatenate([dst, jnp.full((pad,), N, jnp.int32)])
    src_g128, dst_s128 = (a.reshape(NW, EPW // 32, 32)
                          for a in (src_g, dst_s))
    src_g64, dst_s64 = (a.reshape(NW, N_CHUNKS, CHUNK)
                        for a in (src_g, dst_s))

    zeros_deg = jnp.zeros((N_PAD, DEG_D), f32)
    zeros64 = jnp.zeros((N_PAD, 64), f32)
    zeros128 = jnp.zeros((N_PAD, 128), f32)
    ones_deg = jnp.ones((CHUNK, DEG_D), f32)

    degree = _make_degree()
    spmv128 = _make_spmv(128, 32, RING=4, PREF=2)
    spmv64 = _make_spmv(64, 128)

    degp = degree(ones_deg, src_s, zeros_deg)
    dis, s = _tc(_prep_body,
                 (jax.ShapeDtypeStruct((N_PAD, 1), f32),
                  jax.ShapeDtypeStruct((N, 128), f32)),
                 degp, x)

    b0_ = b0.reshape(1, -1)
    b1_ = b1.reshape(1, -1)
    gamma_ = gamma.reshape(1, -1)
    beta_ = beta.reshape(1, -1)

    h = x
    for layer in range(1 + NUM_PROP):
        d_in = 128 if layer == 0 else 64
        spmv = spmv128 if layer == 0 else spmv64
        src_g_r = src_g128 if layer == 0 else src_g64
        dst_s_r = dst_s128 if layer == 0 else dst_s64
        zeros = zeros128 if layer == 0 else zeros64
        W = W0 if layer == 0 else W1
        b_ = b0_ if layer == 0 else b1_
        sds = jax.ShapeDtypeStruct((N, d_in), f32)

        accp = spmv(s, src_g_r, dst_s_r, zeros)
        tx1, s = _tc(_step1_body, (sds, sds), accp, dis)
        accp = spmv(s, src_g_r, dst_s_r, zeros)
        tx2, s = _tc(_stepk_body, (sds, sds), accp, dis, h)
        accp = spmv(s, src_g_r, dst_s_r, zeros)
        tx3, s = _tc(_stepk_body, (sds, sds), accp, dis, tx1)
        accp = spmv(s, src_g_r, dst_s_r, zeros)
        h, s = _tc(_post_body,
                   (jax.ShapeDtypeStruct((N, 64), f32),
                    jax.ShapeDtypeStruct((N, 64), f32)),
                   accp, dis, h, tx1, tx2, tx3, W, b_, gamma_, beta_)

    return _tc(_pool_body, jax.ShapeDtypeStruct((N_GRAPHS, 10), f32),
               h, batch.reshape(1, N), lin2_W, lin2_b.reshape(1, -1))


# speedup vs baseline: 8.9481x; 1.1519x over previous
import functools

import jax
import jax.numpy as jnp
from jax import lax
from jax.experimental import pallas as pl
from jax.experimental.pallas import tpu as pltpu
from jax.experimental.pallas import tpu_sc as plsc

N = 10000
E = 320000
NUM_PROP = 6
N_GRAPHS = 64

NC = 2
NS = 16
NW = NC * NS
CHUNK = 128
N_PAD = 10112
EPW = 10112
E_PAD = NW * EPW
N_CHUNKS = EPW // CHUNK
DEG_D = 16

_MESH = plsc.VectorSubcoreMesh(core_axis_name="c", subcore_axis_name="s")


def _make_spmv(d, chunk, RING=8, PREF=4):
    rows_per_sub = N_PAD // NS
    n_chunks = EPW // chunk

    nfull = n_chunks // RING
    assert nfull >= 2

    @functools.partial(
        pl.kernel, mesh=_MESH,
        compiler_params=pltpu.CompilerParams(use_tc_tiling_on_sc=False),
        out_type=jax.ShapeDtypeStruct((NC, N_PAD, d), jnp.float32),
        scratch_types=[
            pltpu.VMEM((n_chunks, chunk), jnp.int32),
            pltpu.VMEM((n_chunks, chunk), jnp.int32),
        ] + [pltpu.VMEM((chunk, d), jnp.float32)] * RING + [
            pltpu.VMEM_SHARED((N_PAD, d), jnp.float32),
        ] + [pltpu.SemaphoreType.DMA] * (2 * RING),
    )
    def spmv(table, srcs, dsts, zeros, out, src_v, dst_v, *rest):
        bufs = rest[:RING]
        acc = rest[RING]
        sgs = rest[RING + 1:RING + 1 + RING]
        sss = rest[RING + 1 + RING:]
        c = lax.axis_index("c")
        s = lax.axis_index("s")
        w = c * NS + s
        r0 = s * rows_per_sub
        pltpu.sync_copy(zeros.at[pl.ds(r0, rows_per_sub)],
                        acc.at[pl.ds(r0, rows_per_sub)])
        pltpu.sync_copy(srcs.at[w], src_v)
        pltpu.sync_copy(dsts.at[w], dst_v)
        plsc.subcore_barrier()

        def gstart(k, i):
            pltpu.async_copy(table.at[src_v.at[k]], bufs[i], sgs[i])

        def gwait(k, i):
            pltpu.make_async_copy(table.at[src_v.at[k]], bufs[i],
                                  sgs[i]).wait()

        def sstart(k, i):
            pltpu.async_copy(bufs[i], acc.at[dst_v.at[k]], sss[i], add=True)

        def swait(k, i):
            pltpu.make_async_copy(bufs[i], acc.at[dst_v.at[k]],
                                  sss[i]).wait()

        def static_step(k):
            i = k % RING
            gwait(k, i)
            sstart(k, i)
            k2 = k + PREF
            if k2 < n_chunks:
                if k2 - RING >= 0:
                    swait(k2 - RING, k2 % RING)
                gstart(k2, k2 % RING)

        for k in range(PREF):
            gstart(k, k)
        for k in range(RING):
            static_step(k)

        def group(j, carry):
            k0 = RING * j
            for i in range(RING):
                k = k0 + i
                gwait(k, i)
                sstart(k, i)
                swait(k - (RING - PREF), (i + PREF) % RING)
                gstart(k + PREF, (i + PREF) % RING)
            return carry

        lax.fori_loop(1, nfull - 1, group, 0)
        for k in range(RING * (nfull - 1), n_chunks):
            static_step(k)
        for k in range(max(0, n_chunks - RING), n_chunks):
            swait(k, k % RING)
        plsc.subcore_barrier()
        pltpu.sync_copy(acc.at[pl.ds(r0, rows_per_sub)],
                        out.at[c, pl.ds(r0, rows_per_sub)])

    return spmv


def _make_degree():
    rows_per_sub = N_PAD // NS

    @functools.partial(
        pl.kernel, mesh=_MESH,
        compiler_params=pltpu.CompilerParams(use_tc_tiling_on_sc=False),
        out_type=jax.ShapeDtypeStruct((NC, N_PAD, DEG_D), jnp.float32),
        scratch_types=[
            pltpu.VMEM((N_CHUNKS, CHUNK), jnp.int32),
            pltpu.VMEM((CHUNK, DEG_D), jnp.float32),
            pltpu.VMEM_SHARED((N_PAD, DEG_D), jnp.float32),
            pltpu.SemaphoreType.DMA,
        ],
    )
    def degk(ones, idxs, zeros, out, idx_v, ones_v, acc, ssem):
        c = lax.axis_index("c")
        s = lax.axis_index("s")
        w = c * NS + s
        r0 = s * rows_per_sub
        pltpu.sync_copy(zeros.at[pl.ds(r0, rows_per_sub)],
                        acc.at[pl.ds(r0, rows_per_sub)])
        pltpu.sync_copy(idxs.at[w], idx_v)
        pltpu.sync_copy(ones, ones_v)
        plsc.subcore_barrier()

        def body(k, carry):
            pltpu.async_copy(ones_v, acc.at[idx_v.at[k]], ssem, add=True)
            return carry

        lax.fori_loop(0, N_CHUNKS, body, 0)

        def drain(k, carry):
            pltpu.make_async_copy(ones_v, acc.at[idx_v.at[k]], ssem).wait()
            return carry

        lax.fori_loop(0, N_CHUNKS, drain, 0)
        plsc.subcore_barrier()
        pltpu.sync_copy(acc.at[pl.ds(r0, rows_per_sub)],
                        out.at[c, pl.ds(r0, rows_per_sub)])

    return degk


def _prep_body(degp_ref, x_ref, dis_ref, slo_ref, shi_ref):
    d = degp_ref[0, :, 0:1] + degp_ref[1, :, 0:1]
    dis = jnp.where(d > 0.0, lax.rsqrt(jnp.maximum(d, 1.0)), 0.0)
    dis_ref[...] = dis
    s0 = x_ref[...] * dis[:N]
    slo_ref[...] = s0[:, :64]
    shi_ref[...] = s0[:, 64:]


def _step1_body2(acclo_ref, acchi_ref, dis_ref, tx_ref, slo_ref, shi_ref):
    dis = dis_ref[...]
    tlo = -dis * (acclo_ref[0] + acclo_ref[1])
    thi = -dis * (acchi_ref[0] + acchi_ref[1])
    tx_ref[...] = jnp.concatenate([tlo, thi], axis=1)
    slo_ref[...] = dis * tlo
    shi_ref[...] = dis * thi


def _stepk_body2(acclo_ref, acchi_ref, dis_ref, txprev_ref,
                 tx_ref, slo_ref, shi_ref):
    dis = dis_ref[...]
    tlo = -2.0 * dis * (acclo_ref[0] + acclo_ref[1]) - txprev_ref[:, :64]
    thi = -2.0 * dis * (acchi_ref[0] + acchi_ref[1]) - txprev_ref[:, 64:]
    tx_ref[...] = jnp.concatenate([tlo, thi], axis=1)
    slo_ref[...] = dis * tlo
    shi_ref[...] = dis * thi


def _l0mm_body(acclo_ref, acchi_ref, dis_ref, tx0_ref, tx1_ref, tx2_ref,
               tx3_ref, w_ref, b_ref, h0_ref):
    dis = dis_ref[...]
    tlo = -2.0 * dis * (acclo_ref[0] + acclo_ref[1]) - tx2_ref[:, :64]
    thi = -2.0 * dis * (acchi_ref[0] + acchi_ref[1]) - tx2_ref[:, 64:]
    tx4 = jnp.concatenate([tlo, thi], axis=1)
    out = jnp.dot(tx0_ref[...], w_ref[0], preferred_element_type=jnp.float32)
    out += jnp.dot(tx1_ref[...], w_ref[1], preferred_element_type=jnp.float32)
    out += jnp.dot(tx2_ref[...], w_ref[2], preferred_element_type=jnp.float32)
    out += jnp.dot(tx3_ref[...], w_ref[3], preferred_element_type=jnp.float32)
    out += jnp.dot(tx4, w_ref[4], preferred_element_type=jnp.float32)
    h0_ref[...] = jnp.maximum(out + b_ref[...], 0.0)


def _bn_body(h0_ref, dis_ref, gamma_ref, beta_ref, h_ref, s_ref):
    h0 = h0_ref[...]
    mu = jnp.mean(h0, axis=0, keepdims=True)
    hc = h0 - mu
    var = jnp.mean(hc * hc, axis=0, keepdims=True)
    h = hc * lax.rsqrt(var + 1e-5) * gamma_ref[...] + beta_ref[...]
    h_ref[...] = h
    s_ref[...] = dis_ref[...][:N] * h


def _step1_body(accp_ref, dis_ref, tx_ref, s_ref):
    dis = dis_ref[...][:N]
    a = accp_ref[0, :N] + accp_ref[1, :N]
    t = -dis * a
    tx_ref[...] = t
    s_ref[...] = dis * t


def _stepk_body(accp_ref, dis_ref, txprev_ref, tx_ref, s_ref):
    dis = dis_ref[...][:N]
    a = accp_ref[0, :N] + accp_ref[1, :N]
    t = -2.0 * dis * a - txprev_ref[...]
    tx_ref[...] = t
    s_ref[...] = dis * t


def _post_body(accp_ref, dis_ref, tx0_ref, tx1_ref, tx2_ref, tx3_ref,
               w_ref, b_ref, gamma_ref, beta_ref, h_ref, s_ref):
    dis = dis_ref[...][:N]
    tx4 = -2.0 * dis * (accp_ref[0, :N] + accp_ref[1, :N]) - tx2_ref[...]
    out = jnp.dot(tx0_ref[...], w_ref[0], preferred_element_type=jnp.float32)
    out += jnp.dot(tx1_ref[...], w_ref[1], preferred_element_type=jnp.float32)
    out += jnp.dot(tx2_ref[...], w_ref[2], preferred_element_type=jnp.float32)
    out += jnp.dot(tx3_ref[...], w_ref[3], preferred_element_type=jnp.float32)
    out += jnp.dot(tx4, w_ref[4], preferred_element_type=jnp.float32)
    h0 = jnp.maximum(out + b_ref[...], 0.0)
    mu = jnp.mean(h0, axis=0, keepdims=True)
    hc = h0 - mu
    var = jnp.mean(hc * hc, axis=0, keepdims=True)
    h = (h0 - mu) * lax.rsqrt(var + 1e-5) * gamma_ref[...] + beta_ref[...]
    h_ref[...] = h
    s_ref[...] = dis * h


def _pool_body(h_ref, batch_ref, w_ref, b_ref, out_ref):
    gids = lax.broadcasted_iota(jnp.int32, (N_GRAPHS, N), 0)
    m = (gids == batch_ref[...]).astype(jnp.float32)
    sums = jnp.dot(m, h_ref[...], preferred_element_type=jnp.float32)
    counts = jnp.sum(m, axis=1, keepdims=True)
    pooled = sums / jnp.clip(counts, 1.0, None)
    out_ref[...] = (jnp.dot(pooled, w_ref[...],
                            preferred_element_type=jnp.float32) + b_ref[...])


def _tc(body, out_shapes, *args):
    return pl.pallas_call(
        body, out_shape=out_shapes,
        compiler_params=pltpu.CompilerParams(
            vmem_limit_bytes=100 * 1024 * 1024),
    )(*args)


_RB = 2000
_NB = N // _RB

_ACC_SPEC = pl.BlockSpec((NC, _RB, 64), lambda b: (0, b, 0))
_DIS_SPEC = pl.BlockSpec((_RB, 1), lambda b: (b, 0))
_R64_SPEC = pl.BlockSpec((_RB, 64), lambda b: (b, 0))
_R128_SPEC = pl.BlockSpec((_RB, 128), lambda b: (b, 0))


def _tc_grid(body, in_specs, out_specs, out_shapes, *args):
    return pl.pallas_call(
        body, grid=(_NB,), in_specs=list(in_specs),
        out_specs=list(out_specs), out_shape=list(out_shapes),
        compiler_params=pltpu.CompilerParams(
            vmem_limit_bytes=100 * 1024 * 1024),
    )(*args)


def kernel(x, edge_index, batch, W0, b0, W1, b1, gamma, beta, lin2_W, lin2_b):
    f32 = jnp.float32
    src = edge_index[0]
    dst = edge_index[1]
    pad = E_PAD - E
    src_g = jnp.concatenate([src, jnp.zeros((pad,), jnp.int32)])
    src_s = jnp.concatenate([src, jnp.full((pad,), N, jnp.int32)]).reshape(
        NW, N_CHUNKS, CHUNK)
    dst_s = jnp.concatenate([dst, jnp.full((pad,), N, jnp.int32)])
    src_g = src_g.reshape(NW, N_CHUNKS, CHUNK)
    dst_s = dst_s.reshape(NW, N_CHUNKS, CHUNK)

    zeros_deg = jnp.zeros((N_PAD, DEG_D), f32)
    zeros64 = jnp.zeros((N_PAD, 64), f32)
    ones_deg = jnp.ones((CHUNK, DEG_D), f32)

    degree = _make_degree()
    spmv64 = _make_spmv(64, 128, RING=8, PREF=4)

    def spmv(t):
        return spmv64(t, src_g, dst_s, zeros64)

    degp = degree(ones_deg, src_s, zeros_deg)

    sds64 = jax.ShapeDtypeStruct((N, 64), f32)
    sds128 = jax.ShapeDtypeStruct((N, 128), f32)

    dis, slo, shi = _tc(_prep_body,
                        (jax.ShapeDtypeStruct((N_PAD, 1), f32),
                         sds64, sds64),
                        degp, x)

    b0_ = b0.reshape(1, -1)
    b1_ = b1.reshape(1, -1)
    gamma_ = gamma.reshape(1, -1)
    beta_ = beta.reshape(1, -1)

    step2_in = [_ACC_SPEC, _ACC_SPEC, _DIS_SPEC]
    step2_out = [_R128_SPEC, _R64_SPEC, _R64_SPEC]
    acclo, acchi = spmv(slo), spmv(shi)
    tx1, slo, shi = _tc_grid(_step1_body2, step2_in, step2_out,
                             (sds128, sds64, sds64), acclo, acchi, dis)
    acclo, acchi = spmv(slo), spmv(shi)
    tx2, slo, shi = _tc_grid(_stepk_body2, step2_in + [_R128_SPEC],
                             step2_out, (sds128, sds64, sds64),
                             acclo, acchi, dis, x)
    acclo, acchi = spmv(slo), spmv(shi)
    tx3, slo, shi = _tc_grid(_stepk_body2, step2_in + [_R128_SPEC],
                             step2_out, (sds128, sds64, sds64),
                             acclo, acchi, dis, tx1)
    acclo, acchi = spmv(slo), spmv(shi)
    h0, = _tc_grid(
        _l0mm_body,
        step2_in + [_R128_SPEC] * 4 +
        [pl.BlockSpec((5, 128, 64), lambda b: (0, 0, 0)),
         pl.BlockSpec((1, 64), lambda b: (0, 0))],
        [_R64_SPEC], [sds64],
        acclo, acchi, dis, x, tx1, tx2, tx3, W0, b0_)
    h, s = _tc(_bn_body, (sds64, sds64), h0, dis, gamma_, beta_)

    for _ in range(NUM_PROP):
        accp = spmv(s)
        tx1, s = _tc(_step1_body, (sds64, sds64), accp, dis)
        accp = spmv(s)
        tx2, s = _tc(_stepk_body, (sds64, sds64), accp, dis, h)
        accp = spmv(s)
        tx3, s = _tc(_stepk_body, (sds64, sds64), accp, dis, tx1)
        accp = spmv(s)
        h, s = _tc(_post_body, (sds64, sds64),
                   accp, dis, h, tx1, tx2, tx3, W1, b1_, gamma_, beta_)

    return _tc(_pool_body, jax.ShapeDtypeStruct((N_GRAPHS, 10), f32),
               h, batch.reshape(1, N), lin2_W, lin2_b.reshape(1, -1))


# speedup vs baseline: 14.3373x; 1.6023x over previous
import functools

import jax
import jax.numpy as jnp
from jax import lax
from jax.experimental import pallas as pl
from jax.experimental.pallas import tpu as pltpu
from jax.experimental.pallas import tpu_sc as plsc

N = 10000
E = 320000
NUM_PROP = 6
N_GRAPHS = 64

NC = 2
NS = 16
NW = NC * NS
CHUNK = 128
N_PAD = 10112
EPW = 10112
E_PAD = NW * EPW
N_CHUNKS = EPW // CHUNK
DEG_D = 16

_MESH = plsc.VectorSubcoreMesh(core_axis_name="c", subcore_axis_name="s")


def _make_spmv(d, chunk, RING=8, PREF=4):
    rows_per_sub = N_PAD // NS
    n_chunks = EPW // chunk

    nfull = n_chunks // RING
    assert nfull >= 2

    @functools.partial(
        pl.kernel, mesh=_MESH,
        compiler_params=pltpu.CompilerParams(use_tc_tiling_on_sc=False),
        out_type=jax.ShapeDtypeStruct((NC, N_PAD, d), jnp.float32),
        scratch_types=[
            pltpu.VMEM((n_chunks, chunk), jnp.int32),
            pltpu.VMEM((n_chunks, chunk), jnp.int32),
        ] + [pltpu.VMEM((chunk, d), jnp.float32)] * RING + [
            pltpu.VMEM_SHARED((N_PAD, d), jnp.float32),
            pltpu.VMEM_SHARED((N_PAD, d), jnp.float32),
        ] + [pltpu.SemaphoreType.DMA] * (2 * RING),
    )
    def spmv(table, srcs, dsts, zeros, out, src_v, dst_v, *rest):
        bufs = rest[:RING]
        acc = rest[RING]
        table_s = rest[RING + 1]
        sgs = rest[RING + 2:RING + 2 + RING]
        sss = rest[RING + 2 + RING:]
        c = lax.axis_index("c")
        s = lax.axis_index("s")
        w = c * NS + s
        r0 = s * rows_per_sub
        pltpu.sync_copy(zeros.at[pl.ds(r0, rows_per_sub)],
                        acc.at[pl.ds(r0, rows_per_sub)])
        pltpu.sync_copy(table.at[pl.ds(r0, rows_per_sub)],
                        table_s.at[pl.ds(r0, rows_per_sub)])
        pltpu.sync_copy(srcs.at[w], src_v)
        pltpu.sync_copy(dsts.at[w], dst_v)
        plsc.subcore_barrier()

        def gstart(k, i):
            pltpu.async_copy(table_s.at[src_v.at[k]], bufs[i], sgs[i])

        def gwait(k, i):
            pltpu.make_async_copy(table_s.at[src_v.at[k]], bufs[i],
                                  sgs[i]).wait()

        def sstart(k, i):
            pltpu.async_copy(bufs[i], acc.at[dst_v.at[k]], sss[i], add=True)

        def swait(k, i):
            pltpu.make_async_copy(bufs[i], acc.at[dst_v.at[k]],
                                  sss[i]).wait()

        def static_step(k):
            i = k % RING
            gwait(k, i)
            sstart(k, i)
            k2 = k + PREF
            if k2 < n_chunks:
                if k2 - RING >= 0:
                    swait(k2 - RING, k2 % RING)
                gstart(k2, k2 % RING)

        for k in range(PREF):
            gstart(k, k)
        for k in range(RING):
            static_step(k)

        def group(j, carry):
            k0 = RING * j
            for i in range(RING):
                k = k0 + i
                gwait(k, i)
                sstart(k, i)
                swait(k - (RING - PREF), (i + PREF) % RING)
                gstart(k + PREF, (i + PREF) % RING)
            return carry

        lax.fori_loop(1, nfull - 1, group, 0)
        for k in range(RING * (nfull - 1), n_chunks):
            static_step(k)
        for k in range(max(0, n_chunks - RING), n_chunks):
            swait(k, k % RING)
        plsc.subcore_barrier()
        pltpu.sync_copy(acc.at[pl.ds(r0, rows_per_sub)],
                        out.at[c, pl.ds(r0, rows_per_sub)])

    return spmv


def _make_degree():
    rows_per_sub = N_PAD // NS

    @functools.partial(
        pl.kernel, mesh=_MESH,
        compiler_params=pltpu.CompilerParams(use_tc_tiling_on_sc=False),
        out_type=jax.ShapeDtypeStruct((NC, N_PAD, DEG_D), jnp.float32),
        scratch_types=[
            pltpu.VMEM((N_CHUNKS, CHUNK), jnp.int32),
            pltpu.VMEM((CHUNK, DEG_D), jnp.float32),
            pltpu.VMEM_SHARED((N_PAD, DEG_D), jnp.float32),
            pltpu.SemaphoreType.DMA,
        ],
    )
    def degk(ones, idxs, zeros, out, idx_v, ones_v, acc, ssem):
        c = lax.axis_index("c")
        s = lax.axis_index("s")
        w = c * NS + s
        r0 = s * rows_per_sub
        pltpu.sync_copy(zeros.at[pl.ds(r0, rows_per_sub)],
                        acc.at[pl.ds(r0, rows_per_sub)])
        pltpu.sync_copy(idxs.at[w], idx_v)
        pltpu.sync_copy(ones, ones_v)
        plsc.subcore_barrier()

        def body(k, carry):
            pltpu.async_copy(ones_v, acc.at[idx_v.at[k]], ssem, add=True)
            return carry

        lax.fori_loop(0, N_CHUNKS, body, 0)

        def drain(k, carry):
            pltpu.make_async_copy(ones_v, acc.at[idx_v.at[k]], ssem).wait()
            return carry

        lax.fori_loop(0, N_CHUNKS, drain, 0)
        plsc.subcore_barrier()
        pltpu.sync_copy(acc.at[pl.ds(r0, rows_per_sub)],
                        out.at[c, pl.ds(r0, rows_per_sub)])

    return degk


def _prep_body(degp_ref, x_ref, dis_ref, slo_ref, shi_ref):
    d = degp_ref[0, :, 0:1] + degp_ref[1, :, 0:1]
    dis = jnp.where(d > 0.0, lax.rsqrt(jnp.maximum(d, 1.0)), 0.0)
    dis_ref[...] = dis
    s0 = x_ref[...] * dis[:N]
    slo_ref[...] = s0[:, :64]
    shi_ref[...] = s0[:, 64:]


def _step1_body2(acclo_ref, acchi_ref, dis_ref, tx_ref, slo_ref, shi_ref):
    dis = dis_ref[...]
    tlo = -dis * (acclo_ref[0] + acclo_ref[1])
    thi = -dis * (acchi_ref[0] + acchi_ref[1])
    tx_ref[...] = jnp.concatenate([tlo, thi], axis=1)
    slo_ref[...] = dis * tlo
    shi_ref[...] = dis * thi


def _stepk_body2(acclo_ref, acchi_ref, dis_ref, txprev_ref,
                 tx_ref, slo_ref, shi_ref):
    dis = dis_ref[...]
    tlo = -2.0 * dis * (acclo_ref[0] + acclo_ref[1]) - txprev_ref[:, :64]
    thi = -2.0 * dis * (acchi_ref[0] + acchi_ref[1]) - txprev_ref[:, 64:]
    tx_ref[...] = jnp.concatenate([tlo, thi], axis=1)
    slo_ref[...] = dis * tlo
    shi_ref[...] = dis * thi


def _l0mm_body(acclo_ref, acchi_ref, dis_ref, tx0_ref, tx1_ref, tx2_ref,
               tx3_ref, w_ref, b_ref, h0_ref):
    dis = dis_ref[...]
    tlo = -2.0 * dis * (acclo_ref[0] + acclo_ref[1]) - tx2_ref[:, :64]
    thi = -2.0 * dis * (acchi_ref[0] + acchi_ref[1]) - tx2_ref[:, 64:]
    tx4 = jnp.concatenate([tlo, thi], axis=1)
    out = jnp.dot(tx0_ref[...], w_ref[0], preferred_element_type=jnp.float32)
    out += jnp.dot(tx1_ref[...], w_ref[1], preferred_element_type=jnp.float32)
    out += jnp.dot(tx2_ref[...], w_ref[2], preferred_element_type=jnp.float32)
    out += jnp.dot(tx3_ref[...], w_ref[3], preferred_element_type=jnp.float32)
    out += jnp.dot(tx4, w_ref[4], preferred_element_type=jnp.float32)
    h0_ref[...] = jnp.maximum(out + b_ref[...], 0.0)


def _bn_body(h0_ref, dis_ref, gamma_ref, beta_ref, h_ref, s_ref):
    h0 = h0_ref[...]
    mu = jnp.mean(h0, axis=0, keepdims=True)
    hc = h0 - mu
    var = jnp.mean(hc * hc, axis=0, keepdims=True)
    h = hc * lax.rsqrt(var + 1e-5) * gamma_ref[...] + beta_ref[...]
    h_ref[...] = h
    s_ref[...] = dis_ref[...][:N] * h


def _step1_body(accp_ref, dis_ref, tx_ref, s_ref):
    dis = dis_ref[...][:N]
    a = accp_ref[0, :N] + accp_ref[1, :N]
    t = -dis * a
    tx_ref[...] = t
    s_ref[...] = dis * t


def _stepk_body(accp_ref, dis_ref, txprev_ref, tx_ref, s_ref):
    dis = dis_ref[...][:N]
    a = accp_ref[0, :N] + accp_ref[1, :N]
    t = -2.0 * dis * a - txprev_ref[...]
    tx_ref[...] = t
    s_ref[...] = dis * t


def _post_body(accp_ref, dis_ref, tx0_ref, tx1_ref, tx2_ref, tx3_ref,
               w_ref, b_ref, gamma_ref, beta_ref, h_ref, s_ref):
    dis = dis_ref[...][:N]
    tx4 = -2.0 * dis * (accp_ref[0, :N] + accp_ref[1, :N]) - tx2_ref[...]
    out = jnp.dot(tx0_ref[...], w_ref[0], preferred_element_type=jnp.float32)
    out += jnp.dot(tx1_ref[...], w_ref[1], preferred_element_type=jnp.float32)
    out += jnp.dot(tx2_ref[...], w_ref[2], preferred_element_type=jnp.float32)
    out += jnp.dot(tx3_ref[...], w_ref[3], preferred_element_type=jnp.float32)
    out += jnp.dot(tx4, w_ref[4], preferred_element_type=jnp.float32)
    h0 = jnp.maximum(out + b_ref[...], 0.0)
    mu = jnp.mean(h0, axis=0, keepdims=True)
    hc = h0 - mu
    var = jnp.mean(hc * hc, axis=0, keepdims=True)
    h = (h0 - mu) * lax.rsqrt(var + 1e-5) * gamma_ref[...] + beta_ref[...]
    h_ref[...] = h
    s_ref[...] = dis * h


def _pool_body(h_ref, batch_ref, w_ref, b_ref, out_ref):
    gids = lax.broadcasted_iota(jnp.int32, (N_GRAPHS, N), 0)
    m = (gids == batch_ref[...]).astype(jnp.float32)
    sums = jnp.dot(m, h_ref[...], preferred_element_type=jnp.float32)
    counts = jnp.sum(m, axis=1, keepdims=True)
    pooled = sums / jnp.clip(counts, 1.0, None)
    out_ref[...] = (jnp.dot(pooled, w_ref[...],
                            preferred_element_type=jnp.float32) + b_ref[...])


def _tc(body, out_shapes, *args):
    return pl.pallas_call(
        body, out_shape=out_shapes,
        compiler_params=pltpu.CompilerParams(
            vmem_limit_bytes=100 * 1024 * 1024),
    )(*args)


_RB = 2000
_NB = N // _RB

_ACC_SPEC = pl.BlockSpec((NC, _RB, 64), lambda b: (0, b, 0))
_DIS_SPEC = pl.BlockSpec((_RB, 1), lambda b: (b, 0))
_R64_SPEC = pl.BlockSpec((_RB, 64), lambda b: (b, 0))
_R128_SPEC = pl.BlockSpec((_RB, 128), lambda b: (b, 0))


def _tc_grid(body, in_specs, out_specs, out_shapes, *args):
    return pl.pallas_call(
        body, grid=(_NB,), in_specs=list(in_specs),
        out_specs=list(out_specs), out_shape=list(out_shapes),
        compiler_params=pltpu.CompilerParams(
            vmem_limit_bytes=100 * 1024 * 1024),
    )(*args)


def kernel(x, edge_index, batch, W0, b0, W1, b1, gamma, beta, lin2_W, lin2_b):
    f32 = jnp.float32
    src = edge_index[0]
    dst = edge_index[1]
    pad = E_PAD - E
    src_g = jnp.concatenate([src, jnp.zeros((pad,), jnp.int32)])
    src_s = jnp.concatenate([src, jnp.full((pad,), N, jnp.int32)]).reshape(
        NW, N_CHUNKS, CHUNK)
    dst_s = jnp.concatenate([dst, jnp.full((pad,), N, jnp.int32)])
    src_g = src_g.reshape(NW, N_CHUNKS, CHUNK)
    dst_s = dst_s.reshape(NW, N_CHUNKS, CHUNK)

    zeros_deg = jnp.zeros((N_PAD, DEG_D), f32)
    zeros64 = jnp.zeros((N_PAD, 64), f32)
    ones_deg = jnp.ones((CHUNK, DEG_D), f32)

    degree = _make_degree()
    spmv64 = _make_spmv(64, 128, RING=3, PREF=2)

    def spmv(t):
        return spmv64(t, src_g, dst_s, zeros64)

    degp = degree(ones_deg, src_s, zeros_deg)

    sds64 = jax.ShapeDtypeStruct((N, 64), f32)
    sds128 = jax.ShapeDtypeStruct((N, 128), f32)

    dis, slo, shi = _tc(_prep_body,
                        (jax.ShapeDtypeStruct((N_PAD, 1), f32),
                         sds64, sds64),
                        degp, x)

    b0_ = b0.reshape(1, -1)
    b1_ = b1.reshape(1, -1)
    gamma_ = gamma.reshape(1, -1)
    beta_ = beta.reshape(1, -1)

    step2_in = [_ACC_SPEC, _ACC_SPEC, _DIS_SPEC]
    step2_out = [_R128_SPEC, _R64_SPEC, _R64_SPEC]
    acclo, acchi = spmv(slo), spmv(shi)
    tx1, slo, shi = _tc_grid(_step1_body2, step2_in, step2_out,
                             (sds128, sds64, sds64), acclo, acchi, dis)
    acclo, acchi = spmv(slo), spmv(shi)
    tx2, slo, shi = _tc_grid(_stepk_body2, step2_in + [_R128_SPEC],
                             step2_out, (sds128, sds64, sds64),
                             acclo, acchi, dis, x)
    acclo, acchi = spmv(slo), spmv(shi)
    tx3, slo, shi = _tc_grid(_stepk_body2, step2_in + [_R128_SPEC],
                             step2_out, (sds128, sds64, sds64),
                             acclo, acchi, dis, tx1)
    acclo, acchi = spmv(slo), spmv(shi)
    h0, = _tc_grid(
        _l0mm_body,
        step2_in + [_R128_SPEC] * 4 +
        [pl.BlockSpec((5, 128, 64), lambda b: (0, 0, 0)),
         pl.BlockSpec((1, 64), lambda b: (0, 0))],
        [_R64_SPEC], [sds64],
        acclo, acchi, dis, x, tx1, tx2, tx3, W0, b0_)
    h, s = _tc(_bn_body, (sds64, sds64), h0, dis, gamma_, beta_)

    for _ in range(NUM_PROP):
        accp = spmv(s)
        tx1, s = _tc(_step1_body, (sds64, sds64), accp, dis)
        accp = spmv(s)
        tx2, s = _tc(_stepk_body, (sds64, sds64), accp, dis, h)
        accp = spmv(s)
        tx3, s = _tc(_stepk_body, (sds64, sds64), accp, dis, tx1)
        accp = spmv(s)
        h, s = _tc(_post_body, (sds64, sds64),
                   accp, dis, h, tx1, tx2, tx3, W1, b1_, gamma_, beta_)

    return _tc(_pool_body, jax.ShapeDtypeStruct((N_GRAPHS, 10), f32),
               h, batch.reshape(1, N), lin2_W, lin2_b.reshape(1, -1))
